# Initial kernel scaffold; baseline (speedup 1.0000x reference)
import jax
import jax.numpy as jnp
from jax import lax
from jax.experimental import pallas as pl
from jax.experimental.pallas import tpu as pltpu
from jax.experimental.pallas import tpu_sc as plsc

N = 10000
E = 320000
D = 128
H = 128
C = 64

NC = 2
NS = 16
NW = NC * NS
EPT = E // NW
K = 80
CPT = EPT // K
RPT = N // NS
ZR = 25


def _make_sc_aggregate(dp: int, with_counts: bool):
    mesh = plsc.VectorSubcoreMesh(core_axis_name="c", subcore_axis_name="s")
    out_type = [jax.ShapeDtypeStruct((NC * N, dp), jnp.float32)]
    scratch = [
        pltpu.VMEM((CPT, K), jnp.int32),
        pltpu.VMEM((CPT, K), jnp.int32),
        pltpu.VMEM((K, dp), jnp.float32),
        pltpu.VMEM((ZR, dp), jnp.float32),
        pltpu.VMEM_SHARED((N, dp), jnp.float32),
        pltpu.SemaphoreType.DMA,
    ]
    if with_counts:
        out_type.append(jax.ShapeDtypeStruct((NW * N,), jnp.float32))
        scratch.append(pltpu.VMEM((N,), jnp.float32))

    def body(feat_hbm, src_hbm, dst_hbm, out_hbm, *rest):
        if with_counts:
            cnt_hbm, src_loc, dst_loc, rows, zbuf, accum, sem, cnt_loc = rest
        else:
            src_loc, dst_loc, rows, zbuf, accum, sem = rest
        c = lax.axis_index("c")
        s = lax.axis_index("s")
        wid = c * NS + s

        zeros16 = jnp.zeros((16,), jnp.float32)
        for r in range(ZR):
            for c0 in range(dp // 16):
                zbuf[r, c0 * 16:(c0 + 1) * 16] = zeros16

        def zcp(j, carry):
            pltpu.sync_copy(zbuf, accum.at[pl.ds(s * RPT + j * ZR, ZR)])
            return carry
        lax.fori_loop(0, RPT // ZR, zcp, 0)

        if with_counts:
            def zcnt(i, carry):
                cnt_loc[pl.ds(i * 16, 16)] = zeros16
                return carry
            lax.fori_loop(0, N // 16, zcnt, 0)

        pltpu.sync_copy(src_hbm.at[pl.ds(wid * CPT, CPT)], src_loc)
        pltpu.sync_copy(dst_hbm.at[pl.ds(wid * CPT, CPT)], dst_loc)

        plsc.subcore_barrier()

        ones16 = jnp.full((16,), 1.0, jnp.float32)

        def chunk(i, carry):
            pltpu.async_copy(feat_hbm.at[src_loc.at[i]], rows, sem).wait()
            pltpu.sync_copy(rows, accum.at[dst_loc.at[i]], add=True)
            if with_counts:
                for j in range(K // 16):
                    idxv = dst_loc[i, j * 16:(j + 1) * 16]
                    plsc.addupdate_scatter(cnt_loc, [idxv], ones16)
            return carry
        lax.fori_loop(0, CPT, chunk, 0)

        plsc.subcore_barrier()
        pltpu.sync_copy(accum.at[pl.ds(s * RPT, RPT)],
                        out_hbm.at[pl.ds(c * N + s * RPT, RPT)])
        if with_counts:
            pltpu.sync_copy(cnt_loc, cnt_hbm.at[pl.ds(wid * N, N)])

    return pl.kernel(body, out_type=out_type, mesh=mesh,
                     scratch_types=scratch)


_sc_agg_l1 = _make_sc_aggregate(D, True)
_sc_agg_l2 = _make_sc_aggregate(C, False)

RB = 500


def _tc1_body(p0, p1, cnt, x, wl, b1, wr, w2l, h_ref, y2_ref):
    cs = jnp.sum(cnt[...], axis=0)
    inv = 1.0 / jnp.maximum(cs, 1.0)
    mean = (p0[...] + p1[...]) * inv[:, None]
    dims = (((1,), (1,)), ((), ()))
    h = lax.dot_general(mean, wl[...], dims,
                        preferred_element_type=jnp.float32)
    h = h + b1[...] + lax.dot_general(x[...], wr[...], dims,
                                      preferred_element_type=jnp.float32)
    h = jnp.maximum(h, 0.0)
    h_ref[...] = h
    y2_ref[...] = lax.dot_general(h, w2l[...], dims,
                                  preferred_element_type=jnp.float32)


def _tc2_body(q0, q1, cnt, h, wr, b2, out_ref):
    cs = jnp.sum(cnt[...], axis=0)
    inv = 1.0 / jnp.maximum(cs, 1.0)
    dims = (((1,), (1,)), ((), ()))
    o = (q0[...] + q1[...]) * inv[:, None] + b2[...]
    o = o + lax.dot_general(h[...], wr[...], dims,
                            preferred_element_type=jnp.float32)
    m = jnp.max(o, axis=1, keepdims=True)
    e = jnp.exp(o - m)
    lse = jnp.log(jnp.sum(e, axis=1, keepdims=True))
    out_ref[...] = o - m - lse


def _rows_spec(width):
    return pl.BlockSpec((RB, width), lambda i: (i, 0))


def _full_spec(shape):
    return pl.BlockSpec(shape, lambda i: tuple(0 for _ in shape))


_tc1 = pl.pallas_call(
    _tc1_body,
    grid=(N // RB,),
    in_specs=[
        _rows_spec(D), _rows_spec(D),
        pl.BlockSpec((NW, RB), lambda i: (0, i)),
        _rows_spec(D),
        _full_spec((H, D)), _full_spec((1, H)), _full_spec((H, D)),
        _full_spec((C, H)),
    ],
    out_specs=[_rows_spec(H), _rows_spec(C)],
    out_shape=[jax.ShapeDtypeStruct((N, H), jnp.float32),
               jax.ShapeDtypeStruct((N, C), jnp.float32)],
)

_tc2 = pl.pallas_call(
    _tc2_body,
    grid=(N // RB,),
    in_specs=[
        _rows_spec(C), _rows_spec(C),
        pl.BlockSpec((NW, RB), lambda i: (0, i)),
        _rows_spec(H),
        _full_spec((C, H)), _full_spec((1, C)),
    ],
    out_specs=_rows_spec(C),
    out_shape=jax.ShapeDtypeStruct((N, C), jnp.float32),
)


def _as_list(x):
    return list(x) if isinstance(x, (tuple, list)) else [x]


@jax.jit
def kernel(x, edge_index, W1_l, b1_l, W1_r, W2_l, b2_l, W2_r):
    src2d = edge_index[0].reshape(E // K, K)
    dst2d = edge_index[1].reshape(E // K, K)

    sums1, cnt = _as_list(_sc_agg_l1(x, src2d, dst2d))
    cnt = cnt.reshape(NW, N)
    h, y2 = _tc1(sums1[:N], sums1[N:], cnt, x,
                 W1_l, b1_l.reshape(1, H), W1_r, W2_l)
    sums2, = _as_list(_sc_agg_l2(y2, src2d, dst2d))
    out = _tc2(sums2[:N], sums2[N:], cnt, h, W2_r, b2_l.reshape(1, C))
    return out


# speedup vs baseline: 7.9144x; 7.9144x over previous
import functools

import jax
import jax.numpy as jnp
from jax import lax
from jax.experimental import pallas as pl
from jax.experimental.pallas import tpu as pltpu
from jax.experimental.pallas import tpu_sc as plsc

N = 10000
E = 320000
D = 128
H = 128
C = 64

NC = 2
NS = 16
NW = NC * NS
EPT = E // NW
K = 80
CPT = EPT // K
SB = 25
NB = CPT // SB
RPT = 624
TAIL = N - NS * RPT
ZR = 16


def _make_sc_aggregate(dp: int, with_counts: bool):
    mesh = plsc.VectorSubcoreMesh(core_axis_name="c", subcore_axis_name="s",
                                  num_cores=NC, num_subcores=NS)
    out_type = [jax.ShapeDtypeStruct((NC * N, dp), jnp.float32)]
    scratch = [
        pltpu.VMEM((SB, K), jnp.int32),
        pltpu.VMEM((SB, K), jnp.int32),
        pltpu.VMEM((K, dp), jnp.float32),
        pltpu.VMEM((ZR, dp), jnp.float32),
        pltpu.VMEM_SHARED((N, dp), jnp.float32),
        pltpu.SemaphoreType.DMA,
    ]
    if with_counts:
        out_type.append(jax.ShapeDtypeStruct((NC * N, 16), jnp.float32))
        scratch.extend([
            pltpu.VMEM((K, 16), jnp.float32),
            pltpu.VMEM((ZR, 16), jnp.float32),
            pltpu.VMEM_SHARED((N, 16), jnp.float32),
        ])

    def body(feat_hbm, src_hbm, dst_hbm, out_hbm, *rest):
        if with_counts:
            (cnt_hbm, src_loc, dst_loc, rows, zbuf, accum, sem,
             ones_loc, zcnt, cnt_sp) = rest
        else:
            src_loc, dst_loc, rows, zbuf, accum, sem = rest
        c = lax.axis_index("c")
        s = lax.axis_index("s")
        wid = c * NS + s

        zeros16 = jnp.zeros((16,), jnp.float32)
        ones16 = jnp.full((16,), 1.0, jnp.float32)
        for r in range(ZR):
            for c0 in range(dp // 16):
                zbuf[r, c0 * 16:(c0 + 1) * 16] = zeros16
        if with_counts:
            for r in range(ZR):
                zcnt[r, 0:16] = zeros16
            for r in range(K):
                ones_loc[r, 0:16] = ones16

        def zcp(j, carry):
            pltpu.sync_copy(zbuf, accum.at[pl.ds(s * RPT + j * ZR, ZR)])
            if with_counts:
                pltpu.sync_copy(zcnt, cnt_sp.at[pl.ds(s * RPT + j * ZR, ZR)])
            return carry
        lax.fori_loop(0, RPT // ZR, zcp, 0)

        @pl.when(s == NS - 1)
        def _():
            pltpu.sync_copy(zbuf.at[pl.ds(0, TAIL)],
                            accum.at[pl.ds(NS * RPT, TAIL)])
            if with_counts:
                pltpu.sync_copy(zcnt.at[pl.ds(0, TAIL)],
                                cnt_sp.at[pl.ds(NS * RPT, TAIL)])

        plsc.subcore_barrier()

        def block(b, carry):
            pltpu.sync_copy(src_hbm.at[wid, b], src_loc)
            pltpu.sync_copy(dst_hbm.at[wid, b], dst_loc)

            def chunk(i, carry2):
                pltpu.async_copy(feat_hbm.at[src_loc.at[i]], rows, sem).wait()
                pltpu.sync_copy(rows, accum.at[dst_loc.at[i]], add=True)
                if with_counts:
                    pltpu.sync_copy(ones_loc, cnt_sp.at[dst_loc.at[i]],
                                    add=True)
                return carry2
            lax.fori_loop(0, SB, chunk, 0)
            return carry
        lax.fori_loop(0, NB, block, 0)

        plsc.subcore_barrier()
        pltpu.sync_copy(accum.at[pl.ds(s * RPT, RPT)],
                        out_hbm.at[pl.ds(c * N + s * RPT, RPT)])
        if with_counts:
            pltpu.sync_copy(cnt_sp.at[pl.ds(s * RPT, RPT)],
                            cnt_hbm.at[pl.ds(c * N + s * RPT, RPT)])

        @pl.when(s == NS - 1)
        def _():
            pltpu.sync_copy(accum.at[pl.ds(NS * RPT, TAIL)],
                            out_hbm.at[pl.ds(c * N + NS * RPT, TAIL)])
            if with_counts:
                pltpu.sync_copy(cnt_sp.at[pl.ds(NS * RPT, TAIL)],
                                cnt_hbm.at[pl.ds(c * N + NS * RPT, TAIL)])

    params = pltpu.CompilerParams(use_tc_tiling_on_sc=False)
    return pl.kernel(body, out_type=out_type, mesh=mesh,
                     scratch_types=scratch, compiler_params=params)


@functools.lru_cache(maxsize=None)
def _sc_aggregate(dp: int, with_counts: bool):
    return _make_sc_aggregate(dp, with_counts)

RB = 1000


def _tc1_body(p0, p1, c0, c1, x, wl, b1, wr, w2l, h_ref, y2_ref):
    cs = c0[...][:, 0] + c1[...][:, 0]
    inv = 1.0 / jnp.maximum(cs, 1.0)
    mean = (p0[...] + p1[...]) * inv[:, None]
    dims = (((1,), (1,)), ((), ()))
    h = lax.dot_general(mean, wl[...], dims,
                        preferred_element_type=jnp.float32)
    h = h + b1[...] + lax.dot_general(x[...], wr[...], dims,
                                      preferred_element_type=jnp.float32)
    h = jnp.maximum(h, 0.0)
    h_ref[...] = h
    y2_ref[...] = lax.dot_general(h, w2l[...], dims,
                                  preferred_element_type=jnp.float32)


def _tc2_body(q0, q1, c0, c1, h, wr, b2, out_ref):
    cs = c0[...][:, 0] + c1[...][:, 0]
    inv = 1.0 / jnp.maximum(cs, 1.0)
    dims = (((1,), (1,)), ((), ()))
    o = (q0[...] + q1[...]) * inv[:, None] + b2[...]
    o = o + lax.dot_general(h[...], wr[...], dims,
                            preferred_element_type=jnp.float32)
    m = jnp.max(o, axis=1, keepdims=True)
    e = jnp.exp(o - m)
    lse = jnp.log(jnp.sum(e, axis=1, keepdims=True))
    out_ref[...] = o - m - lse


def _rows_spec(width):
    return pl.BlockSpec((RB, width), lambda i: (i, 0))


def _full_spec(shape):
    return pl.BlockSpec(shape, lambda i: tuple(0 for _ in shape))


_tc1 = pl.pallas_call(
    _tc1_body,
    grid=(N // RB,),
    in_specs=[
        _rows_spec(D), _rows_spec(D),
        _rows_spec(16), _rows_spec(16),
        _rows_spec(D),
        _full_spec((H, D)), _full_spec((1, H)), _full_spec((H, D)),
        _full_spec((C, H)),
    ],
    out_specs=[_rows_spec(H), _rows_spec(C)],
    out_shape=[jax.ShapeDtypeStruct((N, H), jnp.float32),
               jax.ShapeDtypeStruct((N, C), jnp.float32)],
)

_tc2 = pl.pallas_call(
    _tc2_body,
    grid=(N // RB,),
    in_specs=[
        _rows_spec(C), _rows_spec(C),
        _rows_spec(16), _rows_spec(16),
        _rows_spec(H),
        _full_spec((C, H)), _full_spec((1, C)),
    ],
    out_specs=_rows_spec(C),
    out_shape=jax.ShapeDtypeStruct((N, C), jnp.float32),
)


def _as_list(x):
    return list(x) if isinstance(x, (tuple, list)) else [x]


@jax.jit
def kernel(x, edge_index, W1_l, b1_l, W1_r, W2_l, b2_l, W2_r):
    src4d = edge_index[0].reshape(NW, NB, SB, K)
    dst4d = edge_index[1].reshape(NW, NB, SB, K)

    sums1, cnt = _as_list(_sc_aggregate(D, True)(x, src4d, dst4d))
    h, y2 = _tc1(sums1[:N], sums1[N:], cnt[:N], cnt[N:], x,
                 W1_l, b1_l.reshape(1, H), W1_r, W2_l)
    sums2, = _as_list(_sc_aggregate(C, False)(y2, src4d, dst4d))
    out = _tc2(sums2[:N], sums2[N:], cnt[:N], cnt[N:], h,
               W2_r, b2_l.reshape(1, C))
    return out


# speedup vs baseline: 11.4493x; 1.4467x over previous
import functools

import jax
import jax.numpy as jnp
from jax import lax
from jax.experimental import pallas as pl
from jax.experimental.pallas import tpu as pltpu
from jax.experimental.pallas import tpu_sc as plsc

N = 10000
E = 320000
D = 128
H = 128
C = 64

NC = 2
NS = 16
NW = NC * NS
EPT = E // NW
K = 80
CPT = EPT // K
SB = 25
NB = CPT // SB
RPT = 624
TAIL = N - NS * RPT
ZR = 16


def _make_sc_aggregate(dp: int, with_counts: bool):
    mesh = plsc.VectorSubcoreMesh(core_axis_name="c", subcore_axis_name="s",
                                  num_cores=NC, num_subcores=NS)
    out_type = [jax.ShapeDtypeStruct((NC * N, dp), jnp.float32)]
    scratch = [
        pltpu.VMEM((SB, K), jnp.int32),
        pltpu.VMEM((SB, K), jnp.int32),
        pltpu.VMEM((K, dp), jnp.float32),
        pltpu.VMEM((K, dp), jnp.float32),
        pltpu.VMEM((ZR, dp), jnp.float32),
        pltpu.VMEM_SHARED((N, dp), jnp.float32),
        pltpu.SemaphoreType.DMA,
        pltpu.SemaphoreType.DMA,
    ]
    if with_counts:
        out_type.append(jax.ShapeDtypeStruct((NC * N, 16), jnp.float32))
        scratch.extend([
            pltpu.VMEM((K, 16), jnp.float32),
            pltpu.VMEM((ZR, 16), jnp.float32),
            pltpu.VMEM_SHARED((N, 16), jnp.float32),
        ])

    def body(feat_hbm, src_hbm, dst_hbm, out_hbm, *rest):
        if with_counts:
            (cnt_hbm, src_loc, dst_loc, rows0, rows1, zbuf, accum,
             sem0, sem1, ones_loc, zcnt, cnt_sp) = rest
        else:
            src_loc, dst_loc, rows0, rows1, zbuf, accum, sem0, sem1 = rest
        c = lax.axis_index("c")
        s = lax.axis_index("s")
        wid = c * NS + s

        zeros16 = jnp.zeros((16,), jnp.float32)
        ones16 = jnp.full((16,), 1.0, jnp.float32)
        for r in range(ZR):
            for c0 in range(dp // 16):
                zbuf[r, c0 * 16:(c0 + 1) * 16] = zeros16
        if with_counts:
            for r in range(ZR):
                zcnt[r, 0:16] = zeros16
            for r in range(K):
                ones_loc[r, 0:16] = ones16

        def zcp(j, carry):
            pltpu.sync_copy(zbuf, accum.at[pl.ds(s * RPT + j * ZR, ZR)])
            if with_counts:
                pltpu.sync_copy(zcnt, cnt_sp.at[pl.ds(s * RPT + j * ZR, ZR)])
            return carry
        lax.fori_loop(0, RPT // ZR, zcp, 0)

        @pl.when(s == NS - 1)
        def _():
            pltpu.sync_copy(zbuf.at[pl.ds(0, TAIL)],
                            accum.at[pl.ds(NS * RPT, TAIL)])
            if with_counts:
                pltpu.sync_copy(zcnt.at[pl.ds(0, TAIL)],
                                cnt_sp.at[pl.ds(NS * RPT, TAIL)])

        plsc.subcore_barrier()

        def block(b, carry):
            pltpu.sync_copy(src_hbm.at[wid, b], src_loc)
            pltpu.sync_copy(dst_hbm.at[wid, b], dst_loc)

            pltpu.async_copy(feat_hbm.at[src_loc.at[0]], rows0, sem0)

            def chunk(i, carry2):
                nxt = i + 1
                @pl.when(jnp.logical_and(nxt < SB, lax.rem(nxt, 2) == 0))
                def _():
                    pltpu.async_copy(feat_hbm.at[src_loc.at[nxt]], rows0,
                                     sem0)

                @pl.when(jnp.logical_and(nxt < SB, lax.rem(nxt, 2) == 1))
                def _():
                    pltpu.async_copy(feat_hbm.at[src_loc.at[nxt]], rows1,
                                     sem1)

                @pl.when(lax.rem(i, 2) == 0)
                def _():
                    pltpu.make_async_copy(feat_hbm.at[src_loc.at[i]], rows0,
                                          sem0).wait()
                    pltpu.sync_copy(rows0, accum.at[dst_loc.at[i]], add=True)

                @pl.when(lax.rem(i, 2) == 1)
                def _():
                    pltpu.make_async_copy(feat_hbm.at[src_loc.at[i]], rows1,
                                          sem1).wait()
                    pltpu.sync_copy(rows1, accum.at[dst_loc.at[i]], add=True)

                if with_counts:
                    pltpu.sync_copy(ones_loc, cnt_sp.at[dst_loc.at[i]],
                                    add=True)
                return carry2
            lax.fori_loop(0, SB, chunk, 0)
            return carry
        lax.fori_loop(0, NB, block, 0)

        plsc.subcore_barrier()
        pltpu.sync_copy(accum.at[pl.ds(s * RPT, RPT)],
                        out_hbm.at[pl.ds(c * N + s * RPT, RPT)])
        if with_counts:
            pltpu.sync_copy(cnt_sp.at[pl.ds(s * RPT, RPT)],
                            cnt_hbm.at[pl.ds(c * N + s * RPT, RPT)])

        @pl.when(s == NS - 1)
        def _():
            pltpu.sync_copy(accum.at[pl.ds(NS * RPT, TAIL)],
                            out_hbm.at[pl.ds(c * N + NS * RPT, TAIL)])
            if with_counts:
                pltpu.sync_copy(cnt_sp.at[pl.ds(NS * RPT, TAIL)],
                                cnt_hbm.at[pl.ds(c * N + NS * RPT, TAIL)])

    params = pltpu.CompilerParams(use_tc_tiling_on_sc=False)
    return pl.kernel(body, out_type=out_type, mesh=mesh,
                     scratch_types=scratch, compiler_params=params)


@functools.lru_cache(maxsize=None)
def _sc_aggregate(dp: int, with_counts: bool):
    return _make_sc_aggregate(dp, with_counts)

RB = 1000


def _tc1_body(p0, p1, c0, c1, x, wl, b1, wr, w2l, h_ref, y2_ref):
    cs = c0[...][:, 0] + c1[...][:, 0]
    inv = 1.0 / jnp.maximum(cs, 1.0)
    mean = (p0[...] + p1[...]) * inv[:, None]
    dims = (((1,), (1,)), ((), ()))
    h = lax.dot_general(mean, wl[...], dims,
                        preferred_element_type=jnp.float32)
    h = h + b1[...] + lax.dot_general(x[...], wr[...], dims,
                                      preferred_element_type=jnp.float32)
    h = jnp.maximum(h, 0.0)
    h_ref[...] = h
    y2_ref[...] = lax.dot_general(h, w2l[...], dims,
                                  preferred_element_type=jnp.float32)


def _tc2_body(q0, q1, c0, c1, h, wr, b2, out_ref):
    cs = c0[...][:, 0] + c1[...][:, 0]
    inv = 1.0 / jnp.maximum(cs, 1.0)
    dims = (((1,), (1,)), ((), ()))
    o = (q0[...] + q1[...]) * inv[:, None] + b2[...]
    o = o + lax.dot_general(h[...], wr[...], dims,
                            preferred_element_type=jnp.float32)
    m = jnp.max(o, axis=1, keepdims=True)
    e = jnp.exp(o - m)
    lse = jnp.log(jnp.sum(e, axis=1, keepdims=True))
    out_ref[...] = o - m - lse


def _rows_spec(width):
    return pl.BlockSpec((RB, width), lambda i: (i, 0))


def _full_spec(shape):
    return pl.BlockSpec(shape, lambda i: tuple(0 for _ in shape))


_tc1 = pl.pallas_call(
    _tc1_body,
    grid=(N // RB,),
    in_specs=[
        _rows_spec(D), _rows_spec(D),
        _rows_spec(16), _rows_spec(16),
        _rows_spec(D),
        _full_spec((H, D)), _full_spec((1, H)), _full_spec((H, D)),
        _full_spec((C, H)),
    ],
    out_specs=[_rows_spec(H), _rows_spec(C)],
    out_shape=[jax.ShapeDtypeStruct((N, H), jnp.float32),
               jax.ShapeDtypeStruct((N, C), jnp.float32)],
)

_tc2 = pl.pallas_call(
    _tc2_body,
    grid=(N // RB,),
    in_specs=[
        _rows_spec(C), _rows_spec(C),
        _rows_spec(16), _rows_spec(16),
        _rows_spec(H),
        _full_spec((C, H)), _full_spec((1, C)),
    ],
    out_specs=_rows_spec(C),
    out_shape=jax.ShapeDtypeStruct((N, C), jnp.float32),
)


def _as_list(x):
    return list(x) if isinstance(x, (tuple, list)) else [x]


@jax.jit
def kernel(x, edge_index, W1_l, b1_l, W1_r, W2_l, b2_l, W2_r):
    src4d = edge_index[0].reshape(NW, NB, SB, K)
    dst4d = edge_index[1].reshape(NW, NB, SB, K)

    sums1, cnt = _as_list(_sc_aggregate(D, True)(x, src4d, dst4d))
    h, y2 = _tc1(sums1[:N], sums1[N:], cnt[:N], cnt[N:], x,
                 W1_l, b1_l.reshape(1, H), W1_r, W2_l)
    sums2, = _as_list(_sc_aggregate(C, False)(y2, src4d, dst4d))
    out = _tc2(sums2[:N], sums2[N:], cnt[:N], cnt[N:], h,
               W2_r, b2_l.reshape(1, C))
    return out
